# baseline (speedup 1.0000x reference)
import functools

import jax
import jax.numpy as jnp
from jax import lax
from jax.experimental import pallas as pl
from jax.experimental.pallas import tpu as pltpu
from jax.experimental.pallas import tpu_sc as plsc

T = 2048
D = 1024
H = 512
E = 64
TB = 256
NTB = T // TB
BLK = 32
MAXT = 128
S = MAXT * BLK
EBN = 256
NC, NS, L = 2, 16, 16
NW = NC * NS

_f32 = jnp.float32
_i32 = jnp.int32


def _silu(z):
    return z * (1.0 / (1.0 + jnp.exp(-z)))


def _route_shared_body(x_ref, gw_ref,
                       aux_ref, dest_ref, eb_ref, src_ref,
                       eid_scr, pos_scr, cnt, accp, accl):
    b = pl.program_id(0)

    @pl.when(b == 0)
    def _():
        cnt[...] = jnp.zeros((1, E), _f32)
        accp[...] = jnp.zeros((1, E), _f32)
        accl[...] = jnp.zeros((1, E), _f32)

    @pl.when(b < NTB)
    def _():
        xb = x_ref[...]
        logits = jnp.dot(xb, gw_ref[...],
                         preferred_element_type=_f32)
        m = jnp.max(logits, axis=1, keepdims=True)
        ex = jnp.exp(logits - m)
        probs = ex / jnp.sum(ex, axis=1, keepdims=True)
        accp[...] = accp[...] + jnp.sum(probs, axis=0, keepdims=True)
        accl[...] = accl[...] + jnp.sum(logits, axis=0, keepdims=True)
        lane = lax.broadcasted_iota(_i32, (TB, E), 1)
        eid = jnp.min(jnp.where(logits == m, lane, E), axis=1)
        oh = (eid[:, None] == lane).astype(_f32)
        ii = lax.broadcasted_iota(_i32, (TB, TB), 0)
        jj = lax.broadcasted_iota(_i32, (TB, TB), 1)
        tri = (jj < ii).astype(_f32)
        excl = jnp.dot(tri, oh, preferred_element_type=_f32)
        posb = jnp.sum(oh * (excl + cnt[...]), axis=1)
        eid_scr[pl.ds(b, 1), :] = eid[None, :].astype(_i32)
        pos_scr[pl.ds(b, 1), :] = posb[None, :].astype(_i32)
        cnt[...] = cnt[...] + jnp.sum(oh, axis=0, keepdims=True)

    @pl.when(b == NTB)
    def _():
        aux_ref[...] = (jnp.sum(accp[...] * accl[...], keepdims=True)
                        * (float(E) / (T * T)))
        cnts = cnt[...]
        aligned = jnp.floor((cnts + (BLK - 1)) * (1.0 / BLK)) * BLK
        ei = lax.broadcasted_iota(_i32, (E, E), 0)
        ej = lax.broadcasted_iota(_i32, (E, E), 1)
        mtx = (ei < ej).astype(_f32)
        excl_al = jnp.dot(aligned, mtx, preferred_element_type=_f32)
        blk_start = excl_al * (1.0 / BLK)
        ntiles = (excl_al[0, E - 1] + aligned[0, E - 1]) * (1.0 / BLK)
        kk = lax.broadcasted_iota(_i32, (EBN, E), 0).astype(_f32)
        eb = jnp.sum((blk_start <= kk).astype(_f32), axis=1) - 1.0
        kvec = lax.broadcasted_iota(_i32, (EBN,), 0)
        eb = jnp.where(kvec == EBN - 1, ntiles, eb)
        eb_ref[...] = eb.astype(_i32)
        lane = lax.broadcasted_iota(_i32, (TB, E), 1)
        for r in range(NTB):
            er = eid_scr[r, :]
            ohr = (er[:, None] == lane).astype(_f32)
            offs = jnp.sum(ohr * excl_al, axis=1)
            dest_ref[r, :] = offs.astype(_i32) + pos_scr[r, :]
        pv0 = lax.broadcasted_iota(_i32, (1, TB), 1)
        ones = jnp.ones((1, TB), _f32)
        for pr in range(S // TB):
            pv = pv0 + pr * TB
            acc = jnp.zeros((1, TB), _f32)
            cov = jnp.zeros((1, TB), _f32)
            for r in range(NTB):
                c = (dest_ref[r, :][:, None] == pv).astype(_f32)
                tv = (pv0 + r * TB).astype(_f32)
                acc = acc + jnp.dot(tv, c, preferred_element_type=_f32)
                cov = cov + jnp.dot(ones, c, preferred_element_type=_f32)
            pad = (pv & (T - 1)).astype(_f32)
            src_ref[pl.ds(pr, 1), :] = jnp.where(cov > 0.0, acc,
                                                 pad).astype(_i32)


def _route(x_flat, gate_w, interpret=False):
    out_shape = [
        jax.ShapeDtypeStruct((1, 1), _f32),
        jax.ShapeDtypeStruct((NTB, TB), _i32),
        jax.ShapeDtypeStruct((EBN,), _i32),
        jax.ShapeDtypeStruct((S // TB, TB), _i32),
    ]
    grid = (NTB + 1,)
    return pl.pallas_call(
        _route_shared_body,
        grid=grid,
        in_specs=[
            pl.BlockSpec((TB, D), lambda b: (jnp.minimum(b, NTB - 1), 0)),
            pl.BlockSpec((D, E), lambda b: (0, 0)),
        ],
        out_specs=[
            pl.BlockSpec((1, 1), lambda b: (0, 0)),
            pl.BlockSpec((NTB, TB), lambda b: (0, 0)),
            pl.BlockSpec((EBN,), lambda b: (0,)),
            pl.BlockSpec((S // TB, TB), lambda b: (0, 0)),
        ],
        out_shape=out_shape,
        scratch_shapes=[
            pltpu.VMEM((NTB, TB), _i32),
            pltpu.VMEM((NTB, TB), _i32),
            pltpu.VMEM((1, E), _f32),
            pltpu.VMEM((1, E), _f32),
            pltpu.VMEM((1, E), _f32),
        ],
        interpret=interpret,
    )(x_flat, gate_w)


def _ffn_body(eb_ref, xp_ref, w1_ref, w2_ref, o_ref):
    k = pl.program_id(0)

    @pl.when(k < eb_ref[EBN - 1])
    def _():
        h = _silu(jnp.dot(xp_ref[...], w1_ref[0], preferred_element_type=_f32))
        o_ref[...] = jnp.dot(h, w2_ref[0], preferred_element_type=_f32)


def _ffn(expert_blk, xp, routed_w1, routed_w2, interpret=False):
    grid_spec = pltpu.PrefetchScalarGridSpec(
        num_scalar_prefetch=1,
        grid=(MAXT,),
        in_specs=[
            pl.BlockSpec((BLK, D), lambda k, eb: (k, 0)),
            pl.BlockSpec((1, D, H), lambda k, eb: (eb[k], 0, 0)),
            pl.BlockSpec((1, H, D), lambda k, eb: (eb[k], 0, 0)),
        ],
        out_specs=pl.BlockSpec((BLK, D), lambda k, eb: (k, 0)),
    )
    return pl.pallas_call(
        _ffn_body,
        grid_spec=grid_spec,
        out_shape=jax.ShapeDtypeStruct((S, D), _f32),
        interpret=interpret,
    )(expert_blk, xp, routed_w1, routed_w2)


def _shared_body(x_ref, sw1_ref, sw2_ref, sh_ref):
    xb = x_ref[...]
    h0 = _silu(jnp.dot(xb, sw1_ref[0], preferred_element_type=_f32))
    h1 = _silu(jnp.dot(xb, sw1_ref[1], preferred_element_type=_f32))
    sh_ref[...] = (jnp.dot(h0, sw2_ref[0], preferred_element_type=_f32)
                   + jnp.dot(h1, sw2_ref[1], preferred_element_type=_f32))


def _shared_ffn(x_flat, shared_w1, shared_w2, interpret=False):
    return pl.pallas_call(
        _shared_body,
        grid=(NTB,),
        in_specs=[
            pl.BlockSpec((TB, D), lambda b: (b, 0)),
            pl.BlockSpec((2, D, H), lambda b: (0, 0, 0)),
            pl.BlockSpec((2, H, D), lambda b: (0, 0, 0)),
        ],
        out_specs=pl.BlockSpec((TB, D), lambda b: (b, 0)),
        out_shape=jax.ShapeDtypeStruct((T, D), _f32),
        interpret=interpret,
    )(x_flat, shared_w1, shared_w2)


_CH = 32
_RPW = S // NW
_NCH = _RPW // _CH


def _sc_gather_body(src_hbm, x_hbm, xp_hbm, idx_all, rows_v, sem):
    wid = lax.axis_index("s") * NC + lax.axis_index("c")
    base = wid * _RPW
    pltpu.sync_copy(src_hbm.at[pl.ds(base, _RPW)], idx_all)
    g = []
    for c in range(3):
        g.append(pltpu.async_copy(
            x_hbm.at[idx_all.at[pl.ds(c * _CH, _CH)]],
            rows_v.at[pl.ds((c % 3) * _CH, _CH)], sem))
    for c in range(_NCH):
        g[c].wait()
        pltpu.sync_copy(rows_v.at[pl.ds((c % 3) * _CH, _CH)],
                        xp_hbm.at[pl.ds(base + c * _CH, _CH)])
        if c + 3 < _NCH:
            g.append(pltpu.async_copy(
                x_hbm.at[idx_all.at[pl.ds((c + 3) * _CH, _CH)]],
                rows_v.at[pl.ds(((c + 3) % 3) * _CH, _CH)], sem))


def _sc_gather(src, x_flat):
    mesh = plsc.VectorSubcoreMesh(core_axis_name="c", subcore_axis_name="s",
                                  num_cores=NC, num_subcores=NS)
    f = functools.partial(
        pl.kernel,
        out_type=jax.ShapeDtypeStruct((S, D), _f32),
        mesh=mesh,
        scratch_types=[
            pltpu.VMEM((_RPW,), _i32),
            pltpu.VMEM((3 * _CH, D), _f32),
            pltpu.SemaphoreType.DMA,
        ],
        compiler_params=pltpu.CompilerParams(needs_layout_passes=False),
    )(_sc_gather_body)
    return f(src, x_flat)


_CB = 32


def _sc_combine_body(dest_hbm, rout_hbm, sh_hbm, y_hbm, idx_v, ra, rb, sem):
    wid = lax.axis_index("s") * NC + lax.axis_index("c")
    base = wid * (T // NW)
    for c in range(T // NW // _CB):
        b = base + c * _CB
        pltpu.sync_copy(dest_hbm.at[pl.ds(b, _CB)], idx_v)
        pltpu.async_copy(rout_hbm.at[idx_v], ra, sem).wait()
        pltpu.sync_copy(sh_hbm.at[pl.ds(b, _CB)], rb)

        def add8(j, carry):
            for u in range(8):
                off = (j * 8 + u) * L
                r = off // D
                o = off % D
                ra[r, pl.ds(o, L)] = ra[r, pl.ds(o, L)] + rb[r, pl.ds(o, L)]
            return carry

        lax.fori_loop(0, _CB * D // L // 8, add8, 0)
        pltpu.sync_copy(ra, y_hbm.at[pl.ds(b, _CB)])


def _sc_combine(dest, rout, shared_sum):
    mesh = plsc.VectorSubcoreMesh(core_axis_name="c", subcore_axis_name="s",
                                  num_cores=NC, num_subcores=NS)
    f = functools.partial(
        pl.kernel,
        out_type=jax.ShapeDtypeStruct((T, D), _f32),
        mesh=mesh,
        scratch_types=[
            pltpu.VMEM((_CB,), _i32),
            pltpu.VMEM((_CB, D), _f32),
            pltpu.VMEM((_CB, D), _f32),
            pltpu.SemaphoreType.DMA,
        ],
        compiler_params=pltpu.CompilerParams(needs_layout_passes=False),
    )(_sc_combine_body)
    return f(dest, rout, shared_sum)


def kernel(x, gate_w, shared_w1, shared_w2, routed_w1, routed_w2):
    x_flat = x.reshape(T, D)
    aux, dest2d, expert_blk, src2d = _route(x_flat, gate_w)
    dest = dest2d.reshape(T)
    xp = _sc_gather(src2d.reshape(S), x_flat)
    shared_sum = _shared_ffn(x_flat, shared_w1, shared_w2)
    rout = _ffn(expert_blk, xp, routed_w1, routed_w2)
    y = _sc_combine(dest, rout, shared_sum)
    return y.reshape(x.shape), aux[0, 0]


# speedup vs baseline: 1.1574x; 1.1574x over previous
import functools

import jax
import jax.numpy as jnp
from jax import lax
from jax.experimental import pallas as pl
from jax.experimental.pallas import tpu as pltpu
from jax.experimental.pallas import tpu_sc as plsc

T = 2048
D = 1024
H = 512
E = 64
TB = 256
NTB = T // TB
BLK = 64
MAXT = 96
S = MAXT * BLK
EBN = 256
NC, NS, L = 2, 16, 16
NW = NC * NS

_f32 = jnp.float32
_i32 = jnp.int32


def _silu(z):
    return z * (1.0 / (1.0 + jnp.exp(-z)))


def _route_shared_body(x_ref, gw_ref,
                       aux_ref, dest_ref, eb_ref, src_ref,
                       eid_scr, pos_scr, cnt, accp, accl):
    b = pl.program_id(0)

    @pl.when(b == 0)
    def _():
        cnt[...] = jnp.zeros((1, E), _f32)
        accp[...] = jnp.zeros((1, E), _f32)
        accl[...] = jnp.zeros((1, E), _f32)

    @pl.when(b < NTB)
    def _():
        xb = x_ref[...]
        logits = jnp.dot(xb, gw_ref[...],
                         preferred_element_type=_f32)
        m = jnp.max(logits, axis=1, keepdims=True)
        ex = jnp.exp(logits - m)
        probs = ex / jnp.sum(ex, axis=1, keepdims=True)
        accp[...] = accp[...] + jnp.sum(probs, axis=0, keepdims=True)
        accl[...] = accl[...] + jnp.sum(logits, axis=0, keepdims=True)
        lane = lax.broadcasted_iota(_i32, (TB, E), 1)
        eid = jnp.min(jnp.where(logits == m, lane, E), axis=1)
        oh = (eid[:, None] == lane).astype(_f32)
        ii = lax.broadcasted_iota(_i32, (TB, TB), 0)
        jj = lax.broadcasted_iota(_i32, (TB, TB), 1)
        tri = (jj < ii).astype(_f32)
        excl = jnp.dot(tri, oh, preferred_element_type=_f32)
        posb = jnp.sum(oh * (excl + cnt[...]), axis=1)
        eid_scr[pl.ds(b, 1), :] = eid[None, :].astype(_i32)
        pos_scr[pl.ds(b, 1), :] = posb[None, :].astype(_i32)
        cnt[...] = cnt[...] + jnp.sum(oh, axis=0, keepdims=True)

    @pl.when(b == NTB)
    def _():
        aux_ref[...] = (jnp.sum(accp[...] * accl[...], keepdims=True)
                        * (float(E) / (T * T)))
        cnts = cnt[...]
        aligned = jnp.floor((cnts + (BLK - 1)) * (1.0 / BLK)) * BLK
        ei = lax.broadcasted_iota(_i32, (E, E), 0)
        ej = lax.broadcasted_iota(_i32, (E, E), 1)
        mtx = (ei < ej).astype(_f32)
        excl_al = jnp.dot(aligned, mtx, preferred_element_type=_f32)
        blk_start = excl_al * (1.0 / BLK)
        ntiles = (excl_al[0, E - 1] + aligned[0, E - 1]) * (1.0 / BLK)
        kk = lax.broadcasted_iota(_i32, (EBN, E), 0).astype(_f32)
        eb = jnp.sum((blk_start <= kk).astype(_f32), axis=1) - 1.0
        kvec = lax.broadcasted_iota(_i32, (EBN,), 0)
        eb = jnp.where(kvec == EBN - 1, ntiles, eb)
        eb_ref[...] = eb.astype(_i32)
        lane = lax.broadcasted_iota(_i32, (TB, E), 1)
        for r in range(NTB):
            er = eid_scr[r, :]
            ohr = (er[:, None] == lane).astype(_f32)
            offs = jnp.sum(ohr * excl_al, axis=1)
            dest_ref[r, :] = offs.astype(_i32) + pos_scr[r, :]
        pv0 = lax.broadcasted_iota(_i32, (1, TB), 1)
        ones = jnp.ones((1, TB), _f32)
        for pr in range(S // TB):
            pv = pv0 + pr * TB
            acc = jnp.zeros((1, TB), _f32)
            cov = jnp.zeros((1, TB), _f32)
            for r in range(NTB):
                c = (dest_ref[r, :][:, None] == pv).astype(_f32)
                tv = (pv0 + r * TB).astype(_f32)
                acc = acc + jnp.dot(tv, c, preferred_element_type=_f32)
                cov = cov + jnp.dot(ones, c, preferred_element_type=_f32)
            pad = (pv & (T - 1)).astype(_f32)
            src_ref[pl.ds(pr, 1), :] = jnp.where(cov > 0.0, acc,
                                                 pad).astype(_i32)


def _route(x_flat, gate_w, interpret=False):
    out_shape = [
        jax.ShapeDtypeStruct((1, 1), _f32),
        jax.ShapeDtypeStruct((NTB, TB), _i32),
        jax.ShapeDtypeStruct((EBN,), _i32),
        jax.ShapeDtypeStruct((S // TB, TB), _i32),
    ]
    grid = (NTB + 1,)
    return pl.pallas_call(
        _route_shared_body,
        grid=grid,
        in_specs=[
            pl.BlockSpec((TB, D), lambda b: (jnp.minimum(b, NTB - 1), 0)),
            pl.BlockSpec((D, E), lambda b: (0, 0)),
        ],
        out_specs=[
            pl.BlockSpec((1, 1), lambda b: (0, 0)),
            pl.BlockSpec((NTB, TB), lambda b: (0, 0)),
            pl.BlockSpec((EBN,), lambda b: (0,)),
            pl.BlockSpec((S // TB, TB), lambda b: (0, 0)),
        ],
        out_shape=out_shape,
        scratch_shapes=[
            pltpu.VMEM((NTB, TB), _i32),
            pltpu.VMEM((NTB, TB), _i32),
            pltpu.VMEM((1, E), _f32),
            pltpu.VMEM((1, E), _f32),
            pltpu.VMEM((1, E), _f32),
        ],
        interpret=interpret,
    )(x_flat, gate_w)


def _ffn_body(eb_ref, xp_ref, w1_ref, w2_ref, o_ref):
    k = pl.program_id(0)

    @pl.when(k < eb_ref[EBN - 1])
    def _():
        h = _silu(jnp.dot(xp_ref[...], w1_ref[0], preferred_element_type=_f32))
        o_ref[...] = jnp.dot(h, w2_ref[0], preferred_element_type=_f32)


def _ffn(expert_blk, xp, routed_w1, routed_w2, interpret=False):
    grid_spec = pltpu.PrefetchScalarGridSpec(
        num_scalar_prefetch=1,
        grid=(MAXT,),
        in_specs=[
            pl.BlockSpec((BLK, D), lambda k, eb: (k, 0)),
            pl.BlockSpec((1, D, H), lambda k, eb: (eb[k], 0, 0)),
            pl.BlockSpec((1, H, D), lambda k, eb: (eb[k], 0, 0)),
        ],
        out_specs=pl.BlockSpec((BLK, D), lambda k, eb: (k, 0)),
    )
    return pl.pallas_call(
        _ffn_body,
        grid_spec=grid_spec,
        out_shape=jax.ShapeDtypeStruct((S, D), _f32),
        interpret=interpret,
    )(expert_blk, xp, routed_w1, routed_w2)


def _shared_body(x_ref, sw1_ref, sw2_ref, sh_ref):
    xb = x_ref[...]
    h0 = _silu(jnp.dot(xb, sw1_ref[0], preferred_element_type=_f32))
    h1 = _silu(jnp.dot(xb, sw1_ref[1], preferred_element_type=_f32))
    sh_ref[...] = (jnp.dot(h0, sw2_ref[0], preferred_element_type=_f32)
                   + jnp.dot(h1, sw2_ref[1], preferred_element_type=_f32))


def _shared_ffn(x_flat, shared_w1, shared_w2, interpret=False):
    return pl.pallas_call(
        _shared_body,
        grid=(NTB,),
        in_specs=[
            pl.BlockSpec((TB, D), lambda b: (b, 0)),
            pl.BlockSpec((2, D, H), lambda b: (0, 0, 0)),
            pl.BlockSpec((2, H, D), lambda b: (0, 0, 0)),
        ],
        out_specs=pl.BlockSpec((TB, D), lambda b: (b, 0)),
        out_shape=jax.ShapeDtypeStruct((T, D), _f32),
        interpret=interpret,
    )(x_flat, shared_w1, shared_w2)


_CH = 32
_RPW = S // NW
_NCH = _RPW // _CH


def _sc_gather_body(src_hbm, x_hbm, xp_hbm, idx_all, rows_v, sem):
    wid = lax.axis_index("s") * NC + lax.axis_index("c")
    base = wid * _RPW
    pltpu.sync_copy(src_hbm.at[pl.ds(base, _RPW)], idx_all)
    g = []
    for c in range(3):
        g.append(pltpu.async_copy(
            x_hbm.at[idx_all.at[pl.ds(c * _CH, _CH)]],
            rows_v.at[pl.ds((c % 3) * _CH, _CH)], sem))
    for c in range(_NCH):
        g[c].wait()
        pltpu.sync_copy(rows_v.at[pl.ds((c % 3) * _CH, _CH)],
                        xp_hbm.at[pl.ds(base + c * _CH, _CH)])
        if c + 3 < _NCH:
            g.append(pltpu.async_copy(
                x_hbm.at[idx_all.at[pl.ds((c + 3) * _CH, _CH)]],
                rows_v.at[pl.ds(((c + 3) % 3) * _CH, _CH)], sem))


def _sc_gather(src, x_flat):
    mesh = plsc.VectorSubcoreMesh(core_axis_name="c", subcore_axis_name="s",
                                  num_cores=NC, num_subcores=NS)
    f = functools.partial(
        pl.kernel,
        out_type=jax.ShapeDtypeStruct((S, D), _f32),
        mesh=mesh,
        scratch_types=[
            pltpu.VMEM((_RPW,), _i32),
            pltpu.VMEM((3 * _CH, D), _f32),
            pltpu.SemaphoreType.DMA,
        ],
        compiler_params=pltpu.CompilerParams(needs_layout_passes=False),
    )(_sc_gather_body)
    return f(src, x_flat)


_CB = 32


def _sc_combine_body(dest_hbm, rout_hbm, sh_hbm, y_hbm, idx_v, ra, rb, sem):
    wid = lax.axis_index("s") * NC + lax.axis_index("c")
    base = wid * (T // NW)
    for c in range(T // NW // _CB):
        b = base + c * _CB
        pltpu.sync_copy(dest_hbm.at[pl.ds(b, _CB)], idx_v)
        pltpu.async_copy(rout_hbm.at[idx_v], ra, sem).wait()
        pltpu.sync_copy(sh_hbm.at[pl.ds(b, _CB)], rb)

        def add8(j, carry):
            for u in range(8):
                off = (j * 8 + u) * L
                r = off // D
                o = off % D
                ra[r, pl.ds(o, L)] = ra[r, pl.ds(o, L)] + rb[r, pl.ds(o, L)]
            return carry

        lax.fori_loop(0, _CB * D // L // 8, add8, 0)
        pltpu.sync_copy(ra, y_hbm.at[pl.ds(b, _CB)])


def _sc_combine(dest, rout, shared_sum):
    mesh = plsc.VectorSubcoreMesh(core_axis_name="c", subcore_axis_name="s",
                                  num_cores=NC, num_subcores=NS)
    f = functools.partial(
        pl.kernel,
        out_type=jax.ShapeDtypeStruct((T, D), _f32),
        mesh=mesh,
        scratch_types=[
            pltpu.VMEM((_CB,), _i32),
            pltpu.VMEM((_CB, D), _f32),
            pltpu.VMEM((_CB, D), _f32),
            pltpu.SemaphoreType.DMA,
        ],
        compiler_params=pltpu.CompilerParams(needs_layout_passes=False),
    )(_sc_combine_body)
    return f(dest, rout, shared_sum)


def kernel(x, gate_w, shared_w1, shared_w2, routed_w1, routed_w2):
    x_flat = x.reshape(T, D)
    aux, dest2d, expert_blk, src2d = _route(x_flat, gate_w)
    dest = dest2d.reshape(T)
    xp = _sc_gather(src2d.reshape(S), x_flat)
    shared_sum = _shared_ffn(x_flat, shared_w1, shared_w2)
    rout = _ffn(expert_blk, xp, routed_w1, routed_w2)
    y = _sc_combine(dest, rout, shared_sum)
    return y.reshape(x.shape), aux[0, 0]
